# baseline (speedup 1.0000x reference)
import functools

import jax
import jax.numpy as jnp
from jax import lax
from jax.experimental import pallas as pl
from jax.experimental.pallas import tpu as pltpu
from jax.experimental.pallas import tpu_sc as plsc

T, D, H, E, K = 2048, 768, 512, 16, 2
TK = T * K
BLK = 256
G = TK // BLK + E
TKP = G * BLK

_NC = 2
_NS = 16
_NW = _NC * _NS
_APW = TK // _NW
_TPW = T // _NW
_LN = 16
_VPR = D // _LN

@functools.lru_cache(maxsize=None)
def _build_sc_dispatch():
    mesh = plsc.VectorSubcoreMesh(core_axis_name="c", subcore_axis_name="s")

    @functools.partial(
        pl.kernel,
        mesh=mesh,
        out_type=jax.ShapeDtypeStruct((TKP, D), jnp.float32),
        scratch_types=[
            pltpu.VMEM((_APW,), jnp.int32),
            pltpu.VMEM((_APW,), jnp.int32),
            pltpu.VMEM((_APW, D), jnp.float32),
            pltpu.SemaphoreType.DMA,
        ],
        compiler_params=pltpu.CompilerParams(needs_layout_passes=False),
    )
    def sc_dispatch(x_hbm, tok_hbm, pos_hbm, xp_hbm, tok_v, pos_v, rows_v, sem):
        wid = lax.axis_index("s") * _NC + lax.axis_index("c")
        base = wid * _APW
        pltpu.sync_copy(tok_hbm.at[pl.ds(base, _APW)], tok_v)
        pltpu.sync_copy(pos_hbm.at[pl.ds(base, _APW)], pos_v)
        pltpu.async_copy(x_hbm.at[tok_v], rows_v, sem).wait()
        pltpu.async_copy(rows_v, xp_hbm.at[pos_v], sem).wait()

    return sc_dispatch


def _sc_dispatch(x, tok, pos):
    return _build_sc_dispatch()(x, tok, pos)


@functools.lru_cache(maxsize=None)
def _build_sc_combine():
    mesh = plsc.VectorSubcoreMesh(core_axis_name="c", subcore_axis_name="s")

    @functools.partial(
        pl.kernel,
        mesh=mesh,
        out_type=jax.ShapeDtypeStruct((T, D), jnp.float32),
        scratch_types=[
            pltpu.VMEM((_APW,), jnp.int32),
            pltpu.VMEM((_APW,), jnp.float32),
            pltpu.VMEM((_APW, D), jnp.float32),
            pltpu.SemaphoreType.DMA,
        ],
        compiler_params=pltpu.CompilerParams(needs_layout_passes=False),
    )
    def sc_combine(outs_hbm, pos_hbm, w_hbm, y_hbm, pos_v, w_v, rows_v, sem):
        wid = lax.axis_index("s") * _NC + lax.axis_index("c")
        base = wid * _APW
        pltpu.sync_copy(pos_hbm.at[pl.ds(base, _APW)], pos_v)
        pltpu.sync_copy(w_hbm.at[pl.ds(base, _APW)], w_v)
        pltpu.async_copy(outs_hbm.at[pos_v], rows_v, sem).wait()

        def body(j, carry):
            w0 = plsc.load_gather(w_v, [jnp.full((_LN,), 2 * j, jnp.int32)])
            w1 = plsc.load_gather(w_v, [jnp.full((_LN,), 2 * j + 1, jnp.int32)])
            for v in range(_VPR):
                s = pl.ds(v * _LN, _LN)
                r0 = rows_v[2 * j, s]
                r1 = rows_v[2 * j + 1, s]
                rows_v[j, s] = r0 * w0 + r1 * w1
            return carry

        lax.fori_loop(0, _TPW, body, 0)
        pltpu.sync_copy(rows_v.at[pl.ds(0, _TPW)],
                        y_hbm.at[pl.ds(wid * _TPW, _TPW)])

    return sc_combine


def _sc_combine(outs, pos, w_flat):
    return _build_sc_combine()(outs, pos, w_flat)


def _ffn_body(em_ref, xm_ref, us_ref, x_ref, wg_ref, wu_ref, wd_ref, o_ref):
    g = pl.program_id(0)

    @pl.when(us_ref[g] == 1)
    def _compute():
        x = x_ref[...]
        gt = jnp.dot(x, wg_ref[0], preferred_element_type=jnp.float32)
        up = jnp.dot(x, wu_ref[0], preferred_element_type=jnp.float32)
        h = (gt * jax.nn.sigmoid(gt)) * up
        o_ref[...] = jnp.dot(h, wd_ref[0], preferred_element_type=jnp.float32)


def _grouped_ffn(emap, xmap, used, xp, Wg, Wu, Wd):
    grid_spec = pltpu.PrefetchScalarGridSpec(
        num_scalar_prefetch=3,
        grid=(G,),
        in_specs=[
            pl.BlockSpec((BLK, D), lambda g, em, xm, us: (xm[g], 0)),
            pl.BlockSpec((1, D, H), lambda g, em, xm, us: (em[g], 0, 0)),
            pl.BlockSpec((1, D, H), lambda g, em, xm, us: (em[g], 0, 0)),
            pl.BlockSpec((1, H, D), lambda g, em, xm, us: (em[g], 0, 0)),
        ],
        out_specs=pl.BlockSpec((BLK, D), lambda g, em, xm, us: (xm[g], 0)),
    )
    return pl.pallas_call(
        _ffn_body,
        grid_spec=grid_spec,
        out_shape=jax.ShapeDtypeStruct((TKP, D), jnp.float32),
    )(emap, xmap, used, xp, Wg, Wu, Wd)


def kernel(x, token_to_expert_indices, weights, Wg, Wu, Wd):
    idx = token_to_expert_indices.astype(jnp.int32)
    flat_e = idx.reshape(TK)
    w_flat = weights.reshape(TK)

    oneh = (flat_e[:, None] == jnp.arange(E, dtype=jnp.int32)[None, :])
    cum = jnp.cumsum(oneh.astype(jnp.int32), axis=0)
    counts = cum[-1]
    rank = jnp.sum(jnp.where(oneh, cum, 0), axis=1) - 1
    nblk = (counts + BLK - 1) // BLK
    blk_cum = jnp.cumsum(nblk)
    bstart = (blk_cum - nblk) * BLK
    pos = jnp.sum(jnp.where(oneh, bstart[None, :], 0), axis=1) + rank

    bid = jnp.arange(G, dtype=jnp.int32)
    nused = blk_cum[-1]
    used = (bid < nused).astype(jnp.int32)
    eb = jnp.sum(bid[:, None] >= blk_cum[None, :], axis=1).astype(jnp.int32)
    eb = jnp.minimum(eb, E - 1)
    last_used = jnp.maximum(nused - 1, 0)
    emap = jnp.where(used == 1, eb, eb[last_used])
    xmap = jnp.minimum(bid, last_used)

    tok = (jnp.arange(TK, dtype=jnp.int32) // K)
    xp = _sc_dispatch(x, tok, pos.astype(jnp.int32))
    outs = _grouped_ffn(emap, xmap, used, xp, Wg, Wu, Wd)
    y = _sc_combine(outs, pos.astype(jnp.int32), w_flat)
    return y
